# Initial kernel scaffold; baseline (speedup 1.0000x reference)
import jax
import jax.numpy as jnp
from jax.experimental import pallas as pl


def kernel(x, edge_index, W_l, b_l, W_r, b_r, gamma, beta):
    raise NotImplementedError("write your pallas kernel here")


# speedup vs baseline: 5.7319x; 5.7319x over previous
import functools

import jax
import jax.numpy as jnp
from jax import lax
from jax.experimental import pallas as pl
from jax.experimental.pallas import tpu as pltpu
from jax.experimental.pallas import tpu_sc as plsc

N_NODES = 10000
N_EDGES = 320000
D = 128
DP = 144
EPS = 1e-5

NUM_CORES = 2
NUM_SUBCORES = 16
NUM_WORKERS = NUM_CORES * NUM_SUBCORES
EDGES_PER_WORKER = N_EDGES // NUM_WORKERS
CHUNK = 80
NUM_CHUNKS = EDGES_PER_WORKER // CHUNK
ROWS_PER_SUBCORE = N_NODES // NUM_SUBCORES


def _sc_agg_kernel(xpad_hbm, src_hbm, dst_hbm, out_hbm, agg_sp, src_v, dst_v,
                   rows_v, sem):
    c = lax.axis_index("c")
    s = lax.axis_index("s")
    wid = s * NUM_CORES + c

    def zrow(r, _):
        def zlane(k, __):
            rows_v[r, pl.ds(k * 16, 16)] = jnp.zeros((16,), jnp.float32)
            return 0
        lax.fori_loop(0, DP // 16, zlane, 0)
        return 0
    lax.fori_loop(0, CHUNK, zrow, 0)

    base_r = s * ROWS_PER_SUBCORE
    n_full = ROWS_PER_SUBCORE // CHUNK
    rem = ROWS_PER_SUBCORE - n_full * CHUNK

    def zcp(j, _):
        pltpu.sync_copy(rows_v, agg_sp.at[pl.ds(base_r + j * CHUNK, CHUNK)])
        return 0
    lax.fori_loop(0, n_full, zcp, 0)
    pltpu.sync_copy(rows_v.at[pl.ds(0, rem)],
                    agg_sp.at[pl.ds(base_r + n_full * CHUNK, rem)])

    plsc.subcore_barrier()

    edge_base = wid * EDGES_PER_WORKER

    def body(i, _):
        base = edge_base + i * CHUNK
        pltpu.sync_copy(src_hbm.at[pl.ds(base, CHUNK)], src_v)
        pltpu.async_copy(xpad_hbm.at[src_v], rows_v, sem).wait()
        pltpu.sync_copy(dst_hbm.at[pl.ds(base, CHUNK)], dst_v)
        pltpu.sync_copy(rows_v, agg_sp.at[dst_v], add=True)
        return 0
    lax.fori_loop(0, NUM_CHUNKS, body, 0)

    plsc.subcore_barrier()

    pltpu.sync_copy(agg_sp.at[pl.ds(base_r, ROWS_PER_SUBCORE)],
                    out_hbm.at[c, pl.ds(base_r, ROWS_PER_SUBCORE)])


@functools.partial(
    pl.kernel,
    mesh=plsc.VectorSubcoreMesh(core_axis_name="c", subcore_axis_name="s"),
    out_type=jax.ShapeDtypeStruct((NUM_CORES, N_NODES, DP), jnp.float32),
    scratch_types=[
        pltpu.VMEM_SHARED((N_NODES, DP), jnp.float32),
        pltpu.VMEM((CHUNK,), jnp.int32),
        pltpu.VMEM((CHUNK,), jnp.int32),
        pltpu.VMEM((CHUNK, DP), jnp.float32),
        pltpu.SemaphoreType.DMA,
    ],
    compiler_params=pltpu.CompilerParams(use_tc_tiling_on_sc=False),
)
def _sc_agg(xpad_hbm, src_hbm, dst_hbm, out_hbm, agg_sp, src_v, dst_v, rows_v,
            sem):
    _sc_agg_kernel(xpad_hbm, src_hbm, dst_hbm, out_hbm, agg_sp, src_v, dst_v,
                   rows_v, sem)


ROW_BLOCK = 1000


def _tc_dense_kernel(agg_ref, x_ref, wl_ref, wr_ref, b_ref, g_ref, be_ref,
                     o_ref):
    a = agg_ref[0] + agg_ref[1]
    feats = a[:, :D]
    cnt = a[:, D:D + 1]
    mean = feats / jnp.maximum(cnt, 1.0)
    x = x_ref[...]
    h = (jnp.dot(mean, wl_ref[...], preferred_element_type=jnp.float32)
         + jnp.dot(x, wr_ref[...], preferred_element_type=jnp.float32)
         + b_ref[...])
    mu = jnp.mean(h, axis=1, keepdims=True)
    var = jnp.mean((h - mu) ** 2, axis=1, keepdims=True)
    h = (h - mu) / jnp.sqrt(var + EPS) * g_ref[...] + be_ref[...]
    o_ref[...] = jnp.maximum(h, 0.0) + x


def _tc_dense(agg2, x, W_l, W_r, b, gamma, beta):
    grid = (N_NODES // ROW_BLOCK,)
    return pl.pallas_call(
        _tc_dense_kernel,
        grid=grid,
        in_specs=[
            pl.BlockSpec((NUM_CORES, ROW_BLOCK, DP), lambda i: (0, i, 0)),
            pl.BlockSpec((ROW_BLOCK, D), lambda i: (i, 0)),
            pl.BlockSpec((D, D), lambda i: (0, 0)),
            pl.BlockSpec((D, D), lambda i: (0, 0)),
            pl.BlockSpec((1, D), lambda i: (0, 0)),
            pl.BlockSpec((1, D), lambda i: (0, 0)),
            pl.BlockSpec((1, D), lambda i: (0, 0)),
        ],
        out_specs=pl.BlockSpec((ROW_BLOCK, D), lambda i: (i, 0)),
        out_shape=jax.ShapeDtypeStruct((N_NODES, D), jnp.float32),
    )(agg2, x, W_l, W_r, b, gamma, beta)


def kernel(x, edge_index, W_l, b_l, W_r, b_r, gamma, beta):
    ei = edge_index.astype(jnp.int32)
    src = ei[0]
    dst = ei[1]
    xpad = jnp.concatenate(
        [x, jnp.ones((N_NODES, 1), x.dtype),
         jnp.zeros((N_NODES, DP - D - 1), x.dtype)], axis=1)
    agg2 = _sc_agg(xpad, src, dst)
    b = (b_l + b_r).reshape(1, D)
    return _tc_dense(agg2, x, W_l, W_r, b, gamma.reshape(1, D),
                     beta.reshape(1, D))


# speedup vs baseline: 11.5587x; 2.0166x over previous
import functools

import jax
import jax.numpy as jnp
from jax import lax
from jax.experimental import pallas as pl
from jax.experimental.pallas import tpu as pltpu
from jax.experimental.pallas import tpu_sc as plsc

N_NODES = 10000
N_EDGES = 320000
D = 128
DP = 144
EPS = 1e-5

NUM_CORES = 2
NUM_SUBCORES = 16
NUM_WORKERS = NUM_CORES * NUM_SUBCORES
EDGES_PER_WORKER = N_EDGES // NUM_WORKERS
CHUNK = 80
NUM_CHUNKS = EDGES_PER_WORKER // CHUNK
NBUF = 3
NBUFI = 5
ROWS_PER_SUBCORE = N_NODES // NUM_SUBCORES


def _sc_agg_kernel(xpad_hbm, eidx_hbm, out_hbm, agg_sp, idx_ring, rows_v,
                   sem_g, sem_s, sem_x):
    c = lax.axis_index("c")
    s = lax.axis_index("s")
    wid = s * NUM_CORES + c
    chunk_base = wid * NUM_CHUNKS

    def start_idx(i, bi):
        pltpu.async_copy(eidx_hbm.at[chunk_base + i], idx_ring.at[bi],
                         sem_x.at[bi])

    def wait_idx(i, bi):
        pltpu.make_async_copy(eidx_hbm.at[chunk_base + i], idx_ring.at[bi],
                              sem_x.at[bi]).wait()

    def start_gather(i, b):
        pltpu.async_copy(xpad_hbm.at[idx_ring.at[i % NBUFI, 0]],
                         rows_v.at[b], sem_g.at[b])

    def wait_gather(i, b):
        pltpu.make_async_copy(xpad_hbm.at[idx_ring.at[i % NBUFI, 0]],
                              rows_v.at[b], sem_g.at[b]).wait()

    def start_scatter(i, b):
        pltpu.async_copy(rows_v.at[b], agg_sp.at[idx_ring.at[i % NBUFI, 1]],
                         sem_s.at[b], add=True)

    def wait_scatter(i, b):
        pltpu.make_async_copy(rows_v.at[b],
                              agg_sp.at[idx_ring.at[i % NBUFI, 1]],
                              sem_s.at[b]).wait()

    for i in range(4):
        start_idx(i, i)

    def zrow(r, _):
        def zlane(k, __):
            rows_v[0, r, pl.ds(k * 16, 16)] = jnp.zeros((16,), jnp.float32)
            return 0
        lax.fori_loop(0, DP // 16, zlane, 0)
        return 0
    lax.fori_loop(0, CHUNK, zrow, 0)

    base_r = s * ROWS_PER_SUBCORE
    n_full = ROWS_PER_SUBCORE // CHUNK
    rem = ROWS_PER_SUBCORE - n_full * CHUNK

    def zcp(j, _):
        pltpu.sync_copy(rows_v.at[0],
                        agg_sp.at[pl.ds(base_r + j * CHUNK, CHUNK)])
        return 0
    lax.fori_loop(0, n_full, zcp, 0)
    pltpu.sync_copy(rows_v.at[0, pl.ds(0, rem)],
                    agg_sp.at[pl.ds(base_r + n_full * CHUNK, rem)])

    plsc.subcore_barrier()

    for i in range(2):
        wait_idx(i, i)
        start_gather(i, i % NBUF)

    def step(i, b):
        wait_gather(i, b)
        start_scatter(i, b)

        @pl.when(i + 2 < NUM_CHUNKS)
        def _():
            bb = (b + 2) % NBUF

            @pl.when(i >= 1)
            def _():
                wait_scatter(i - 1, bb)
            wait_idx(i + 2, (i + 2) % NBUFI)
            start_gather(i + 2, bb)

        @pl.when(i + 4 < NUM_CHUNKS)
        def _():
            start_idx(i + 4, (i + 4) % NBUFI)

    def outer(j, _):
        for b in range(NBUF):
            step(j * NBUF + b, b)
        return 0
    n_main = NUM_CHUNKS // NBUF
    lax.fori_loop(0, n_main, outer, 0)

    for i in range(n_main * NBUF, NUM_CHUNKS):
        step(i, i % NBUF)
    for i in range(NUM_CHUNKS - NBUF, NUM_CHUNKS):
        wait_scatter(i, i % NBUF)

    plsc.subcore_barrier()

    pltpu.sync_copy(agg_sp.at[pl.ds(base_r, ROWS_PER_SUBCORE)],
                    out_hbm.at[c, pl.ds(base_r, ROWS_PER_SUBCORE)])


@functools.partial(
    pl.kernel,
    mesh=plsc.VectorSubcoreMesh(core_axis_name="c", subcore_axis_name="s"),
    out_type=jax.ShapeDtypeStruct((NUM_CORES, N_NODES, DP), jnp.float32),
    scratch_types=[
        pltpu.VMEM_SHARED((N_NODES, DP), jnp.float32),
        pltpu.VMEM((NBUFI, 2, CHUNK), jnp.int32),
        pltpu.VMEM((NBUF, CHUNK, DP), jnp.float32),
        pltpu.SemaphoreType.DMA((NBUF,)),
        pltpu.SemaphoreType.DMA((NBUF,)),
        pltpu.SemaphoreType.DMA((NBUFI,)),
    ],
    compiler_params=pltpu.CompilerParams(use_tc_tiling_on_sc=False),
)
def _sc_agg(xpad_hbm, eidx_hbm, out_hbm, agg_sp, idx_ring, rows_v, sem_g,
            sem_s, sem_x):
    _sc_agg_kernel(xpad_hbm, eidx_hbm, out_hbm, agg_sp, idx_ring, rows_v,
                   sem_g, sem_s, sem_x)


ROW_BLOCK = 1000


def _tc_dense_kernel(agg_ref, x_ref, wl_ref, wr_ref, b_ref, g_ref, be_ref,
                     o_ref):
    a = agg_ref[0] + agg_ref[1]
    feats = a[:, :D]
    cnt = a[:, D:D + 1]
    mean = feats / jnp.maximum(cnt, 1.0)
    x = x_ref[...]
    h = (jnp.dot(mean, wl_ref[...], preferred_element_type=jnp.float32)
         + jnp.dot(x, wr_ref[...], preferred_element_type=jnp.float32)
         + b_ref[...])
    mu = jnp.mean(h, axis=1, keepdims=True)
    var = jnp.mean((h - mu) ** 2, axis=1, keepdims=True)
    h = (h - mu) / jnp.sqrt(var + EPS) * g_ref[...] + be_ref[...]
    o_ref[...] = jnp.maximum(h, 0.0) + x


def _tc_dense(agg2, x, W_l, W_r, b, gamma, beta):
    grid = (N_NODES // ROW_BLOCK,)
    return pl.pallas_call(
        _tc_dense_kernel,
        grid=grid,
        in_specs=[
            pl.BlockSpec((NUM_CORES, ROW_BLOCK, DP), lambda i: (0, i, 0)),
            pl.BlockSpec((ROW_BLOCK, D), lambda i: (i, 0)),
            pl.BlockSpec((D, D), lambda i: (0, 0)),
            pl.BlockSpec((D, D), lambda i: (0, 0)),
            pl.BlockSpec((1, D), lambda i: (0, 0)),
            pl.BlockSpec((1, D), lambda i: (0, 0)),
            pl.BlockSpec((1, D), lambda i: (0, 0)),
        ],
        out_specs=pl.BlockSpec((ROW_BLOCK, D), lambda i: (i, 0)),
        out_shape=jax.ShapeDtypeStruct((N_NODES, D), jnp.float32),
    )(agg2, x, W_l, W_r, b, gamma, beta)


def kernel(x, edge_index, W_l, b_l, W_r, b_r, gamma, beta):
    ei = edge_index.astype(jnp.int32)
    eidx = jnp.transpose(
        ei.reshape(2, N_EDGES // CHUNK, CHUNK), (1, 0, 2))
    xpad = jnp.concatenate(
        [x, jnp.ones((N_NODES, 1), x.dtype),
         jnp.zeros((N_NODES, DP - D - 1), x.dtype)], axis=1)
    agg2 = _sc_agg(xpad, eidx)
    b = (b_l + b_r).reshape(1, D)
    return _tc_dense(agg2, x, W_l, W_r, b, gamma.reshape(1, D),
                     beta.reshape(1, D))
